# baseline (speedup 1.0000x reference)
import jax
import jax.numpy as jnp
from jax.experimental import pallas as pl
from jax.experimental.pallas import tpu as pltpu

N, D, H, Z2 = 4096, 128, 128, 64
BM = 256
LOG2E = 1.4426950408889634
NEGC = -1000000000000.0 * LOG2E


def _leaky(v):
    return jnp.maximum(v, 0.25 * v)


def _mm_kernel(x_ref, w_ref, o_ref):
    o_ref[:] = jnp.dot(x_ref[:], w_ref[:], preferred_element_type=jnp.float32)


def _mm(x, w):
    m, k = x.shape
    _, n = w.shape
    bm = 512
    return pl.pallas_call(
        _mm_kernel,
        grid=(m // bm,),
        in_specs=[
            pl.BlockSpec((bm, k), lambda i: (i, 0)),
            pl.BlockSpec((k, n), lambda i: (0, 0)),
        ],
        out_specs=pl.BlockSpec((bm, n), lambda i: (i, 0)),
        out_shape=jax.ShapeDtypeStruct((m, n), jnp.float32),
    )(x, w)


def _gcn1_kernel(adj_ref, s_ref, b_ref, w_ref, o_ref, mask_ref):
    adj = adj_ref[:]
    acc = jnp.dot(adj, s_ref[:], preferred_element_type=jnp.float32)
    t = _leaky(acc + b_ref[:])
    o_ref[:] = jnp.dot(t, w_ref[:], preferred_element_type=jnp.float32)
    mask_ref[:] = (adj > 0).astype(jnp.int8)


def _gcn1(adj, support, b, w_next):
    h = support.shape[1]
    hn = w_next.shape[1]
    return pl.pallas_call(
        _gcn1_kernel,
        grid=(N // BM,),
        in_specs=[
            pl.BlockSpec((BM, N), lambda i: (i, 0)),
            pl.BlockSpec((N, h), lambda i: (0, 0)),
            pl.BlockSpec((1, h), lambda i: (0, 0)),
            pl.BlockSpec((h, hn), lambda i: (0, 0)),
        ],
        out_specs=[
            pl.BlockSpec((BM, hn), lambda i: (i, 0)),
            pl.BlockSpec((BM, N), lambda i: (i, 0)),
        ],
        out_shape=[
            jax.ShapeDtypeStruct((N, hn), jnp.float32),
            jax.ShapeDtypeStruct((N, N), jnp.int8),
        ],
    )(adj, support, b, w_next)


def _gcn2_kernel(adj_ref, s_ref, b_ref, w_ref, o_ref):
    acc = jnp.dot(adj_ref[:], s_ref[:], preferred_element_type=jnp.float32)
    t = _leaky(acc + b_ref[:])
    o_ref[:] = jnp.dot(t, w_ref[:], preferred_element_type=jnp.float32)


def _gcn2(adj, support, b, w_next):
    h = support.shape[1]
    hn = w_next.shape[1]
    return pl.pallas_call(
        _gcn2_kernel,
        grid=(N // BM,),
        in_specs=[
            pl.BlockSpec((BM, N), lambda i: (i, 0)),
            pl.BlockSpec((N, h), lambda i: (0, 0)),
            pl.BlockSpec((1, h), lambda i: (0, 0)),
            pl.BlockSpec((h, hn), lambda i: (0, 0)),
        ],
        out_specs=pl.BlockSpec((BM, hn), lambda i: (i, 0)),
        out_shape=jax.ShapeDtypeStruct((N, hn), jnp.float32),
    )(adj, support, b, w_next)


def _gat_kernel(mask_ref, h_ref, a1_ref, a2_ref, o_ref):
    i = pl.program_id(0)
    hfull = h_ref[:]
    hb = h_ref[pl.ds(i * BM, BM), :]
    s1 = jnp.sum(hb * a1_ref[:], axis=1, keepdims=True)
    s2 = jnp.sum(hfull * a2_ref[:], axis=1)
    e = _leaky(s1 + s2[None, :])
    e = jnp.where(mask_ref[:].astype(jnp.float32) > 0.0, e, NEGC)
    m = jnp.max(e, axis=1, keepdims=True)
    p = jnp.exp2(e - m)
    l = jnp.sum(p, axis=1, keepdims=True)
    o = jnp.dot(p, hfull, preferred_element_type=jnp.float32) / l
    o_ref[:] = _leaky(o)


def _gat(mask, h, a1, a2):
    return pl.pallas_call(
        _gat_kernel,
        grid=(N // BM,),
        in_specs=[
            pl.BlockSpec((BM, N), lambda i: (i, 0)),
            pl.BlockSpec((N, Z2), lambda i: (0, 0)),
            pl.BlockSpec((1, Z2), lambda i: (0, 0)),
            pl.BlockSpec((1, Z2), lambda i: (0, 0)),
        ],
        out_specs=pl.BlockSpec((BM, Z2), lambda i: (i, 0)),
        out_shape=jax.ShapeDtypeStruct((N, Z2), jnp.float32),
    )(mask, h, a1, a2)


def kernel(x, adj, W1, b1, W2, b2, Wg, a):
    b1r = b1.reshape(1, H)
    b2r = b2.reshape(1, H)
    a1r = (a[:Z2, 0] * LOG2E).reshape(1, Z2)
    a2r = (a[Z2:, 0] * LOG2E).reshape(1, Z2)
    support1 = _mm(x, W1)
    support2, mask = _gcn1(adj, support1, b1r, W2)
    h = _gcn2(adj, support2, b2r, Wg)
    out = _gat(mask, h, a1r, a2r)
    return out[:, : Z2 // 2], out[:, Z2 // 2 :]


# speedup vs baseline: 1.0746x; 1.0746x over previous
import jax
import jax.numpy as jnp
from jax import lax
from jax.experimental import pallas as pl

N, D, H, Z2 = 4096, 128, 128, 64
BM = 256
LOG2E = 1.4426950408889634
NEGC = -1000000000000.0 * LOG2E


def _leaky(v):
    return jnp.maximum(v, 0.25 * v)


def _mm_kernel(x_ref, w_ref, o_ref):
    o_ref[:] = jnp.dot(x_ref[:], w_ref[:], preferred_element_type=jnp.float32)


def _mm(x, w):
    m, k = x.shape
    _, n = w.shape
    bm = 1024
    return pl.pallas_call(
        _mm_kernel,
        grid=(m // bm,),
        in_specs=[
            pl.BlockSpec((bm, k), lambda i: (i, 0)),
            pl.BlockSpec((k, n), lambda i: (0, 0)),
        ],
        out_specs=pl.BlockSpec((bm, n), lambda i: (i, 0)),
        out_shape=jax.ShapeDtypeStruct((m, n), jnp.float32),
    )(x, w)


def _gcn1_kernel(adj_ref, s_ref, b_ref, w_ref, o_ref):
    acc = jnp.dot(adj_ref[:], s_ref[:], preferred_element_type=jnp.float32)
    t = _leaky(acc + b_ref[:])
    o_ref[:] = jnp.dot(t, w_ref[:], preferred_element_type=jnp.float32)


def _gcn1(adj, support, b, w_next):
    h = support.shape[1]
    hn = w_next.shape[1]
    return pl.pallas_call(
        _gcn1_kernel,
        grid=(N // BM,),
        in_specs=[
            pl.BlockSpec((BM, N), lambda i: (i, 0)),
            pl.BlockSpec((N, h), lambda i: (0, 0)),
            pl.BlockSpec((1, h), lambda i: (0, 0)),
            pl.BlockSpec((h, hn), lambda i: (0, 0)),
        ],
        out_specs=pl.BlockSpec((BM, hn), lambda i: (i, 0)),
        out_shape=jax.ShapeDtypeStruct((N, hn), jnp.float32),
    )(adj, support, b, w_next)


def _gcn2_kernel(adj_ref, s_ref, b_ref, w_ref, a1_ref, a2_ref,
                 h_ref, s1_ref, s2t_ref):
    acc = jnp.dot(adj_ref[:], s_ref[:], preferred_element_type=jnp.float32)
    t = _leaky(acc + b_ref[:])
    hb = jnp.dot(t, w_ref[:], preferred_element_type=jnp.float32)
    h_ref[:] = hb
    s1_ref[:] = jnp.dot(hb, a1_ref[:], preferred_element_type=jnp.float32)
    s2t_ref[:] = lax.dot_general(
        a2_ref[:], hb, (((1,), (1,)), ((), ())),
        preferred_element_type=jnp.float32)


def _gcn2(adj, support, b, w_next, a1c, a2r):
    h = support.shape[1]
    hn = w_next.shape[1]
    return pl.pallas_call(
        _gcn2_kernel,
        grid=(N // BM,),
        in_specs=[
            pl.BlockSpec((BM, N), lambda i: (i, 0)),
            pl.BlockSpec((N, h), lambda i: (0, 0)),
            pl.BlockSpec((1, h), lambda i: (0, 0)),
            pl.BlockSpec((h, hn), lambda i: (0, 0)),
            pl.BlockSpec((hn, 1), lambda i: (0, 0)),
            pl.BlockSpec((1, hn), lambda i: (0, 0)),
        ],
        out_specs=[
            pl.BlockSpec((BM, hn), lambda i: (i, 0)),
            pl.BlockSpec((BM, 1), lambda i: (i, 0)),
            pl.BlockSpec((1, BM), lambda i: (0, i)),
        ],
        out_shape=[
            jax.ShapeDtypeStruct((N, hn), jnp.float32),
            jax.ShapeDtypeStruct((N, 1), jnp.float32),
            jax.ShapeDtypeStruct((1, N), jnp.float32),
        ],
    )(adj, support, b, w_next, a1c, a2r)


def _gat_kernel(adj_ref, h_ref, s1_ref, s2t_ref, o_ref):
    e = _leaky(s1_ref[:] + s2t_ref[:])
    e = jnp.where(adj_ref[:] > 0, e, NEGC)
    m = jnp.max(e, axis=1, keepdims=True)
    p = jnp.exp2(e - m)
    l = jnp.sum(p, axis=1, keepdims=True)
    o = jnp.dot(p, h_ref[:], preferred_element_type=jnp.float32) / l
    o_ref[:] = _leaky(o)


def _gat(adj, h, s1, s2t):
    return pl.pallas_call(
        _gat_kernel,
        grid=(N // BM,),
        in_specs=[
            pl.BlockSpec((BM, N), lambda i: (i, 0)),
            pl.BlockSpec((N, Z2), lambda i: (0, 0)),
            pl.BlockSpec((BM, 1), lambda i: (i, 0)),
            pl.BlockSpec((1, N), lambda i: (0, 0)),
        ],
        out_specs=pl.BlockSpec((BM, Z2), lambda i: (i, 0)),
        out_shape=jax.ShapeDtypeStruct((N, Z2), jnp.float32),
    )(adj, h, s1, s2t)


def kernel(x, adj, W1, b1, W2, b2, Wg, a):
    b1r = b1.reshape(1, H)
    b2r = b2.reshape(1, H)
    a1c = (a[:Z2] * LOG2E).reshape(Z2, 1)
    a2r = (a[Z2:, 0] * LOG2E).reshape(1, Z2)
    support1 = _mm(x, W1)
    support2 = _gcn1(adj, support1, b1r, W2)
    h, s1, s2t = _gcn2(adj, support2, b2r, Wg, a1c, a2r)
    out = _gat(adj, h, s1, s2t)
    return out[:, : Z2 // 2], out[:, Z2 // 2 :]
